# baseline (speedup 1.0000x reference)
import functools
import jax
import jax.numpy as jnp
from jax import lax
from jax.experimental import pallas as pl
from jax.experimental.pallas import tpu as pltpu
from jax.experimental.pallas import tpu_sc as plsc

_B = 16384
_F = 26
_VOCAB = 100000
_DIM = 32
_LANES = 16
_NC = 2
_NS = 16
_NW = _NC * _NS
_BPW = _B // _NW


def _sc_body(idx_hbm, tab_hbm, out_hbm,
             idx_v, fidx0, fidx1, buf0, buf1, acc_v, sem0, sem1):
    wid = lax.axis_index("s") * _NC + lax.axis_index("c")
    base = wid * _BPW

    pltpu.sync_copy(idx_hbm.at[:, pl.ds(base, _BPW)], idx_v)

    fidxs = (fidx0, fidx1)
    bufs = (buf0, buf1)
    sems = (sem0, sem1)

    def build(f):
        fidx = fidxs[f % 2]
        off = jnp.int32(f * _VOCAB)

        @pl.loop(0, _BPW // _LANES)
        def _(i):
            s = pl.ds(i * _LANES, _LANES)
            fidx[s] = idx_v[f, s] + off

    def fire(f):
        p = f % 2
        return pltpu.async_copy(tab_hbm.at[fidxs[p]], bufs[p], sems[p])

    def accum(f, copy_desc):
        copy_desc.wait()
        buf = bufs[f % 2]
        if f == 0:
            @pl.loop(0, _BPW, unroll=8)
            def _(j):
                acc_v[j, pl.ds(0, _LANES)] = buf[j, pl.ds(0, _LANES)]
                acc_v[j, pl.ds(_LANES, _LANES)] = buf[j, pl.ds(_LANES, _LANES)]
        else:
            @pl.loop(0, _BPW, unroll=8)
            def _(j):
                plsc.addupdate(acc_v.at[j, pl.ds(0, _LANES)],
                               buf[j, pl.ds(0, _LANES)])
                plsc.addupdate(acc_v.at[j, pl.ds(_LANES, _LANES)],
                               buf[j, pl.ds(_LANES, _LANES)])

    build(0)
    d = {0: fire(0)}
    build(1)
    d[1] = fire(1)
    for f in range(_F):
        accum(f, d[f % 2])
        nf = f + 2
        if nf < _F:
            build(nf)
            d[nf % 2] = fire(nf)

    pltpu.sync_copy(acc_v, out_hbm.at[pl.ds(base, _BPW)])


@jax.jit
def kernel(inputs, tables):
    idx_t = jnp.asarray(inputs, dtype=jnp.int32).T
    tab_flat = tables.reshape(_F * _VOCAB, _DIM)
    mesh = plsc.VectorSubcoreMesh(core_axis_name="c", subcore_axis_name="s")
    run = pl.kernel(
        _sc_body,
        out_type=jax.ShapeDtypeStruct((_B, _DIM), jnp.float32),
        mesh=mesh,
        compiler_params=pltpu.CompilerParams(use_tc_tiling_on_sc=False),
        scratch_types=[
            pltpu.VMEM((_F, _BPW), jnp.int32),
            pltpu.VMEM((_BPW,), jnp.int32),
            pltpu.VMEM((_BPW,), jnp.int32),
            pltpu.VMEM((_BPW, _DIM), jnp.float32),
            pltpu.VMEM((_BPW, _DIM), jnp.float32),
            pltpu.VMEM((_BPW, _DIM), jnp.float32),
            pltpu.SemaphoreType.DMA,
            pltpu.SemaphoreType.DMA,
        ],
    )
    return run(idx_t, tab_flat)


# speedup vs baseline: 1.0025x; 1.0025x over previous
import functools
import jax
import jax.numpy as jnp
from jax import lax
from jax.experimental import pallas as pl
from jax.experimental.pallas import tpu as pltpu
from jax.experimental.pallas import tpu_sc as plsc

_B = 16384
_F = 26
_VOCAB = 100000
_DIM = 32
_LANES = 16
_NC = 2
_NS = 16
_NW = _NC * _NS
_BPW = _B // _NW


def _sc_body(idx_hbm, tab_hbm, out_hbm,
             idx_v, buf0, buf1, acc_v, sem0, sem1):
    wid = lax.axis_index("s") * _NC + lax.axis_index("c")
    base = wid * _BPW

    pltpu.sync_copy(idx_hbm.at[:, pl.ds(base, _BPW)], idx_v)

    bufs = (buf0, buf1)
    sems = (sem0, sem1)

    def fire(f):
        p = f % 2
        return pltpu.async_copy(tab_hbm.at[f].at[idx_v.at[f]], bufs[p], sems[p])

    def accum(f, copy_desc):
        copy_desc.wait()
        buf = bufs[f % 2]
        if f == 0:
            @pl.loop(0, _BPW, unroll=8)
            def _(j):
                acc_v[j, pl.ds(0, _LANES)] = buf[j, pl.ds(0, _LANES)]
                acc_v[j, pl.ds(_LANES, _LANES)] = buf[j, pl.ds(_LANES, _LANES)]
        else:
            @pl.loop(0, _BPW, unroll=8)
            def _(j):
                plsc.addupdate(acc_v.at[j, pl.ds(0, _LANES)],
                               buf[j, pl.ds(0, _LANES)])
                plsc.addupdate(acc_v.at[j, pl.ds(_LANES, _LANES)],
                               buf[j, pl.ds(_LANES, _LANES)])

    d = {0: fire(0), 1: fire(1)}
    for f in range(_F):
        accum(f, d[f % 2])
        nf = f + 2
        if nf < _F:
            d[nf % 2] = fire(nf)

    pltpu.sync_copy(acc_v, out_hbm.at[pl.ds(base, _BPW)])


@jax.jit
def kernel(inputs, tables):
    idx_t = jnp.asarray(inputs, dtype=jnp.int32).T
    mesh = plsc.VectorSubcoreMesh(core_axis_name="c", subcore_axis_name="s")
    run = pl.kernel(
        _sc_body,
        out_type=jax.ShapeDtypeStruct((_B, _DIM), jnp.float32),
        mesh=mesh,
        compiler_params=pltpu.CompilerParams(use_tc_tiling_on_sc=False),
        scratch_types=[
            pltpu.VMEM((_F, _BPW), jnp.int32),
            pltpu.VMEM((_BPW, _DIM), jnp.float32),
            pltpu.VMEM((_BPW, _DIM), jnp.float32),
            pltpu.VMEM((_BPW, _DIM), jnp.float32),
            pltpu.SemaphoreType.DMA,
            pltpu.SemaphoreType.DMA,
        ],
    )
    return run(idx_t, tables)
